# baseline (speedup 1.0000x reference)
import functools

import jax
import jax.numpy as jnp
from jax import lax
from jax.experimental import pallas as pl
from jax.experimental.pallas import tpu as pltpu
from jax.experimental.pallas import tpu_sc as plsc

N = 10000
E = 320000
D = 128

NC = 2
NS = 16
NW = NC * NS

C = 128
CH0 = 160
CH1 = 0
TOT = NS * (CH0 + CH1)
E_PAD = TOT * C
SEGC = 16
SEG0 = CH0 // SEGC
SEG1 = CH1 // SEGC
STRIP = 624
FCHUNK = 104
NFLUSH = STRIP // FCHUNK
TAIL = N - NS * STRIP


def _splat16(v):
    return jnp.full((16,), v, dtype=jnp.int32)


def _sc_agg_kernel(src_hbm, dst_hbm, ew_hbm, x_hbm, out_hbm,
                   src_v, dst_v, ew_v, rows_a, rows_b, agg_sh,
                   sem_ga, sem_gb, sem_sa, sem_sb):
    c = lax.axis_index("c")
    s = lax.axis_index("s")
    rows_v = rows_a

    zeros16 = jnp.zeros((16,), dtype=jnp.float32)

    def _zero_row(e, _):
        for k in range(D // 16):
            rows_v[e, pl.ds(k * 16, 16)] = zeros16
        return 0

    base = s * STRIP

    @pl.when(c == 0)
    def _zero_phase():
        lax.fori_loop(0, C, _zero_row, 0)
        for i in range(NFLUSH):
            pltpu.sync_copy(rows_v.at[pl.ds(0, FCHUNK)],
                            agg_sh.at[pl.ds(base + i * FCHUNK, FCHUNK)])

        @pl.when(s == NS - 1)
        def _zero_tail():
            pltpu.sync_copy(rows_v.at[pl.ds(0, TAIL)],
                            agg_sh.at[pl.ds(NS * STRIP, TAIL)])

    plsc.subcore_barrier()

    def _scale_buf(buf, j):
        def _scale(g, _):
            wv16 = ew_v[j, pl.ds(g * 16, 16)]
            for l in range(16):
                w = wv16[l]
                e = g * 16 + l
                for k in range(D // 16):
                    sl = pl.ds(k * 16, 16)
                    buf[e, sl] = buf[e, sl] * w
            return 0

        lax.fori_loop(0, C // 16, _scale, 0)

    def _gstart(j, buf, sem):
        pltpu.async_copy(x_hbm.at[src_v.at[j]], buf, sem)

    def _gwait(j, buf, sem):
        pltpu.make_async_copy(x_hbm.at[src_v.at[j]], buf, sem).wait()

    def _sstart(j, buf, sem):
        pltpu.async_copy(buf, agg_sh.at[dst_v.at[j]], sem, add=True)

    def _swait(j, buf, sem):
        pltpu.make_async_copy(buf, agg_sh.at[dst_v.at[j]], sem).wait()

    NB2 = SEGC // 2

    tile_base = jnp.where(c == 0, s * CH0, NS * CH0 + s * CH1)
    nseg = jnp.where(c == 0, SEG0, SEG1)

    def _seg(sg, _):
        off = tile_base + sg * SEGC
        pltpu.sync_copy(src_hbm.at[pl.ds(off, SEGC)], src_v)
        pltpu.sync_copy(dst_hbm.at[pl.ds(off, SEGC)], dst_v)
        pltpu.sync_copy(ew_hbm.at[pl.ds(off, SEGC)], ew_v)
        _gstart(0, rows_a, sem_ga)

        def _pair(i, _):
            ja = 2 * i
            jb = 2 * i + 1
            _gwait(ja, rows_a, sem_ga)
            _gstart(jb, rows_b, sem_gb)
            _scale_buf(rows_a, ja)
            pltpu.sync_copy(rows_a, agg_sh.at[dst_v.at[ja]], add=True)
            _gwait(jb, rows_b, sem_gb)

            @pl.when(i < NB2 - 1)
            def _():
                _gstart(ja + 2, rows_a, sem_ga)

            _scale_buf(rows_b, jb)
            pltpu.sync_copy(rows_b, agg_sh.at[dst_v.at[jb]], add=True)
            return 0

        lax.fori_loop(0, NB2, _pair, 0)
        return 0

    lax.fori_loop(0, nseg, _seg, 0)

    plsc.subcore_barrier()

    @pl.when(c == 0)
    def _flush_phase():
        for i in range(NFLUSH):
            sl = pl.ds(base + i * FCHUNK, FCHUNK)
            buf, sg, ss = ((rows_a, sem_ga, sem_sa) if i % 2 == 0
                           else (rows_b, sem_gb, sem_sb))
            bsl = buf.at[pl.ds(0, FCHUNK)]
            if i >= 2:
                psl = pl.ds(base + (i - 2) * FCHUNK, FCHUNK)
                pltpu.make_async_copy(bsl, out_hbm.at[psl], ss).wait()
            pltpu.async_copy(agg_sh.at[sl], bsl, sg).wait()
            pltpu.async_copy(bsl, out_hbm.at[sl], ss)
        for i in (NFLUSH - 2, NFLUSH - 1):
            sl = pl.ds(base + i * FCHUNK, FCHUNK)
            buf, ss = (rows_a, sem_sa) if i % 2 == 0 else (rows_b, sem_sb)
            pltpu.make_async_copy(buf.at[pl.ds(0, FCHUNK)],
                                  out_hbm.at[sl], ss).wait()

        @pl.when(s == NS - 1)
        def _flush_tail():
            sl = pl.ds(NS * STRIP, TAIL)
            tsl = rows_a.at[pl.ds(0, TAIL)]
            pltpu.sync_copy(agg_sh.at[sl], tsl)
            pltpu.sync_copy(tsl, out_hbm.at[sl])


def _tc_finish_kernel(p_ref, w_ref, b_ref, z_ref, z1_ref, z2_ref):
    h = jnp.dot(p_ref[...], w_ref[...], preferred_element_type=jnp.float32)
    b = b_ref[...]
    z_ref[...] = jnp.maximum(h + b, 0.0)
    z1_ref[...] = jnp.maximum(h * 0.9 + b, 0.0)
    z2_ref[...] = jnp.maximum(h * 0.8 + b, 0.0)


def kernel(x, edge_index, edge_weight, W, b):
    pad = E_PAD - E
    src = jnp.pad(edge_index[0], (0, pad)).reshape(TOT, C)
    dst = jnp.pad(edge_index[1], (0, pad)).reshape(TOT, C)
    ew = jnp.pad(edge_weight, (0, pad)).reshape(TOT, C)

    mesh = plsc.VectorSubcoreMesh(core_axis_name="c", subcore_axis_name="s")
    sc_call = pl.kernel(
        _sc_agg_kernel,
        out_type=jax.ShapeDtypeStruct((N, D), jnp.float32),
        mesh=mesh,
        scratch_types=[
            pltpu.VMEM((SEGC, C), jnp.int32),
            pltpu.VMEM((SEGC, C), jnp.int32),
            pltpu.VMEM((SEGC, C), jnp.float32),
            pltpu.VMEM((C, D), jnp.float32),
            pltpu.VMEM((C, D), jnp.float32),
            pltpu.VMEM_SHARED((N, D), jnp.float32),
            pltpu.SemaphoreType.DMA,
            pltpu.SemaphoreType.DMA,
            pltpu.SemaphoreType.DMA,
            pltpu.SemaphoreType.DMA,
        ],
    )
    partials = sc_call(src, dst, ew, x)

    R = 1000
    grid = (N // R,)
    z, z1, z2 = pl.pallas_call(
        _tc_finish_kernel,
        grid=grid,
        in_specs=[
            pl.BlockSpec((R, D), lambda i: (i, 0)),
            pl.BlockSpec((D, D), lambda i: (0, 0)),
            pl.BlockSpec((1, D), lambda i: (0, 0)),
        ],
        out_specs=[
            pl.BlockSpec((R, D), lambda i: (i, 0)),
            pl.BlockSpec((R, D), lambda i: (i, 0)),
            pl.BlockSpec((R, D), lambda i: (i, 0)),
        ],
        out_shape=[jax.ShapeDtypeStruct((N, D), jnp.float32)] * 3,
    )(partials, W, b.reshape(1, D))
    return (z, z1, z2)


# speedup vs baseline: 1.6199x; 1.6199x over previous
import functools

import jax
import jax.numpy as jnp
from jax import lax
from jax.experimental import pallas as pl
from jax.experimental.pallas import tpu as pltpu
from jax.experimental.pallas import tpu_sc as plsc

N = 10000
E = 320000
D = 128

NC = 2
NS = 16
NW = NC * NS

C = 128
CH0 = 136
CH1 = 24
TOT = NS * (CH0 + CH1)
E_PAD = TOT * C
SEGC = 8
SEG0 = CH0 // SEGC
SEG1 = CH1 // SEGC
STRIP = 624
FCHUNK = 104
NFLUSH = STRIP // FCHUNK
TAIL = N - NS * STRIP


def _splat16(v):
    return jnp.full((16,), v, dtype=jnp.int32)


def _sc_agg_kernel(src_hbm, dst_hbm, ew_hbm, x_hbm, out_hbm,
                   src_v, dst_v, ew_v, rows_a, rows_b, agg_sh,
                   sem_ga, sem_gb, sem_sa, sem_sb):
    c = lax.axis_index("c")
    s = lax.axis_index("s")
    rows_v = rows_a

    zeros16 = jnp.zeros((16,), dtype=jnp.float32)

    def _zero_row(e, _):
        for k in range(D // 16):
            rows_v[e, pl.ds(k * 16, 16)] = zeros16
        return 0

    lax.fori_loop(0, C, _zero_row, 0)
    base = s * STRIP
    for i in range(NFLUSH):
        pltpu.sync_copy(rows_v.at[pl.ds(0, FCHUNK)],
                        agg_sh.at[pl.ds(base + i * FCHUNK, FCHUNK)])

    @pl.when(s == NS - 1)
    def _zero_tail():
        pltpu.sync_copy(rows_v.at[pl.ds(0, TAIL)],
                        agg_sh.at[pl.ds(NS * STRIP, TAIL)])

    plsc.subcore_barrier()

    def _scale_buf(buf, j):
        def _scale(g, _):
            wv16 = ew_v[j, pl.ds(g * 16, 16)]
            for l in range(16):
                w = wv16[l]
                e = g * 16 + l
                for k in range(D // 16):
                    sl = pl.ds(k * 16, 16)
                    buf[e, sl] = buf[e, sl] * w
            return 0

        lax.fori_loop(0, C // 16, _scale, 0)

    def _gstart(j, buf, sem):
        pltpu.async_copy(x_hbm.at[src_v.at[j]], buf, sem)

    def _gwait(j, buf, sem):
        pltpu.make_async_copy(x_hbm.at[src_v.at[j]], buf, sem).wait()

    def _sstart(j, buf, sem):
        pltpu.async_copy(buf, agg_sh.at[dst_v.at[j]], sem, add=True)

    def _swait(j, buf, sem):
        pltpu.make_async_copy(buf, agg_sh.at[dst_v.at[j]], sem).wait()

    NB2 = SEGC // 2

    tile_base = jnp.where(c == 0, s * CH0, NS * CH0 + s * CH1)
    nseg = jnp.where(c == 0, SEG0, SEG1)

    def _seg(sg, _):
        off = tile_base + sg * SEGC
        pltpu.sync_copy(src_hbm.at[pl.ds(off, SEGC)], src_v)
        pltpu.sync_copy(dst_hbm.at[pl.ds(off, SEGC)], dst_v)
        pltpu.sync_copy(ew_hbm.at[pl.ds(off, SEGC)], ew_v)
        _gstart(0, rows_a, sem_ga)

        def _pair(i, _):
            ja = 2 * i
            jb = 2 * i + 1
            _gwait(ja, rows_a, sem_ga)

            @pl.when(i > 0)
            def _():
                _swait(jb - 2, rows_b, sem_sb)

            _gstart(jb, rows_b, sem_gb)
            _scale_buf(rows_a, ja)
            _sstart(ja, rows_a, sem_sa)
            _gwait(jb, rows_b, sem_gb)
            _scale_buf(rows_b, jb)
            _swait(ja, rows_a, sem_sa)

            @pl.when(i < NB2 - 1)
            def _():
                _gstart(ja + 2, rows_a, sem_ga)

            _sstart(jb, rows_b, sem_sb)
            return 0

        lax.fori_loop(0, NB2, _pair, 0)
        _swait(SEGC - 1, rows_b, sem_sb)
        return 0

    lax.fori_loop(0, nseg, _seg, 0)

    plsc.subcore_barrier()

    for i in range(NFLUSH):
        sl = pl.ds(base + i * FCHUNK, FCHUNK)
        buf, sg, ss = ((rows_a, sem_ga, sem_sa) if i % 2 == 0
                       else (rows_b, sem_gb, sem_sb))
        bsl = buf.at[pl.ds(0, FCHUNK)]
        if i >= 2:
            psl = pl.ds(base + (i - 2) * FCHUNK, FCHUNK)
            pltpu.make_async_copy(bsl, out_hbm.at[c].at[psl], ss).wait()
        pltpu.async_copy(agg_sh.at[sl], bsl, sg).wait()
        pltpu.async_copy(bsl, out_hbm.at[c].at[sl], ss)
    for i in (NFLUSH - 2, NFLUSH - 1):
        sl = pl.ds(base + i * FCHUNK, FCHUNK)
        buf, ss = (rows_a, sem_sa) if i % 2 == 0 else (rows_b, sem_sb)
        pltpu.make_async_copy(buf.at[pl.ds(0, FCHUNK)],
                              out_hbm.at[c].at[sl], ss).wait()

    @pl.when(s == NS - 1)
    def _flush_tail():
        sl = pl.ds(NS * STRIP, TAIL)
        tsl = rows_a.at[pl.ds(0, TAIL)]
        pltpu.sync_copy(agg_sh.at[sl], tsl)
        pltpu.sync_copy(tsl, out_hbm.at[c].at[sl])


def _tc_finish_kernel(p_ref, w_ref, b_ref, z_ref, z1_ref, z2_ref):
    agg = p_ref[0] + p_ref[1]
    h = jnp.dot(agg, w_ref[...], preferred_element_type=jnp.float32)
    b = b_ref[...]
    z_ref[...] = jnp.maximum(h + b, 0.0)
    z1_ref[...] = jnp.maximum(h * 0.9 + b, 0.0)
    z2_ref[...] = jnp.maximum(h * 0.8 + b, 0.0)


def kernel(x, edge_index, edge_weight, W, b):
    pad = E_PAD - E
    src = jnp.pad(edge_index[0], (0, pad)).reshape(TOT, C)
    dst = jnp.pad(edge_index[1], (0, pad)).reshape(TOT, C)
    ew = jnp.pad(edge_weight, (0, pad)).reshape(TOT, C)

    mesh = plsc.VectorSubcoreMesh(core_axis_name="c", subcore_axis_name="s")
    sc_call = pl.kernel(
        _sc_agg_kernel,
        out_type=jax.ShapeDtypeStruct((NC, N, D), jnp.float32),
        mesh=mesh,
        scratch_types=[
            pltpu.VMEM((SEGC, C), jnp.int32),
            pltpu.VMEM((SEGC, C), jnp.int32),
            pltpu.VMEM((SEGC, C), jnp.float32),
            pltpu.VMEM((C, D), jnp.float32),
            pltpu.VMEM((C, D), jnp.float32),
            pltpu.VMEM_SHARED((N, D), jnp.float32),
            pltpu.SemaphoreType.DMA,
            pltpu.SemaphoreType.DMA,
            pltpu.SemaphoreType.DMA,
            pltpu.SemaphoreType.DMA,
        ],
    )
    partials = sc_call(src, dst, ew, x)

    R = 1000
    grid = (N // R,)
    z, z1, z2 = pl.pallas_call(
        _tc_finish_kernel,
        grid=grid,
        in_specs=[
            pl.BlockSpec((NC, R, D), lambda i: (0, i, 0)),
            pl.BlockSpec((D, D), lambda i: (0, 0)),
            pl.BlockSpec((1, D), lambda i: (0, 0)),
        ],
        out_specs=[
            pl.BlockSpec((R, D), lambda i: (i, 0)),
            pl.BlockSpec((R, D), lambda i: (i, 0)),
            pl.BlockSpec((R, D), lambda i: (i, 0)),
        ],
        out_shape=[jax.ShapeDtypeStruct((N, D), jnp.float32)] * 3,
    )(partials, W, b.reshape(1, D))
    return (z, z1, z2)


# speedup vs baseline: 1.8338x; 1.1321x over previous
import functools

import jax
import jax.numpy as jnp
from jax import lax
from jax.experimental import pallas as pl
from jax.experimental.pallas import tpu as pltpu
from jax.experimental.pallas import tpu_sc as plsc

N = 10000
E = 320000
D = 128

NC = 2
NS = 16
NW = NC * NS

C = 128
CH0 = 144
CH1 = 16
TOT = NS * (CH0 + CH1)
E_PAD = TOT * C
SEGC = 8
SEG0 = CH0 // SEGC
SEG1 = CH1 // SEGC
STRIP = 624
FCHUNK = 104
NFLUSH = STRIP // FCHUNK
TAIL = N - NS * STRIP


def _splat16(v):
    return jnp.full((16,), v, dtype=jnp.int32)


def _sc_agg_kernel(src_hbm, dst_hbm, ew_hbm, x_hbm, out_hbm,
                   src_v, dst_v, ew_v, rows_a, rows_b, agg_sh,
                   sem_ga, sem_gb, sem_sa, sem_sb):
    c = lax.axis_index("c")
    s = lax.axis_index("s")
    rows_v = rows_a

    zeros16 = jnp.zeros((16,), dtype=jnp.float32)

    def _zero_row(e, _):
        for k in range(D // 16):
            rows_v[e, pl.ds(k * 16, 16)] = zeros16
        return 0

    lax.fori_loop(0, C, _zero_row, 0)
    base = s * STRIP
    for i in range(NFLUSH):
        pltpu.sync_copy(rows_v.at[pl.ds(0, FCHUNK)],
                        agg_sh.at[pl.ds(base + i * FCHUNK, FCHUNK)])

    @pl.when(s == NS - 1)
    def _zero_tail():
        pltpu.sync_copy(rows_v.at[pl.ds(0, TAIL)],
                        agg_sh.at[pl.ds(NS * STRIP, TAIL)])

    plsc.subcore_barrier()

    def _scale_buf(buf, j):
        def _scale(g, _):
            wv16 = ew_v[j, pl.ds(g * 16, 16)]
            for l in range(16):
                w = wv16[l]
                e = g * 16 + l
                for k in range(D // 16):
                    sl = pl.ds(k * 16, 16)
                    buf[e, sl] = buf[e, sl] * w
            return 0

        lax.fori_loop(0, C // 16, _scale, 0)

    def _gstart(j, buf, sem):
        pltpu.async_copy(x_hbm.at[src_v.at[j]], buf, sem)

    def _gwait(j, buf, sem):
        pltpu.make_async_copy(x_hbm.at[src_v.at[j]], buf, sem).wait()

    def _sstart(j, buf, sem):
        pltpu.async_copy(buf, agg_sh.at[dst_v.at[j]], sem, add=True)

    def _swait(j, buf, sem):
        pltpu.make_async_copy(buf, agg_sh.at[dst_v.at[j]], sem).wait()

    NB2 = SEGC // 2

    tile_base = jnp.where(c == 0, s * CH0, NS * CH0 + s * CH1)
    nseg = jnp.where(c == 0, SEG0, SEG1)

    def _seg(sg, _):
        off = tile_base + sg * SEGC
        pltpu.sync_copy(src_hbm.at[pl.ds(off, SEGC)], src_v)
        pltpu.sync_copy(dst_hbm.at[pl.ds(off, SEGC)], dst_v)
        pltpu.sync_copy(ew_hbm.at[pl.ds(off, SEGC)], ew_v)
        _gstart(0, rows_a, sem_ga)

        def _pair(i, _):
            ja = 2 * i
            jb = 2 * i + 1
            _gwait(ja, rows_a, sem_ga)

            @pl.when(i > 0)
            def _():
                _swait(jb - 2, rows_b, sem_sb)

            _gstart(jb, rows_b, sem_gb)
            _scale_buf(rows_a, ja)
            _sstart(ja, rows_a, sem_sa)
            _gwait(jb, rows_b, sem_gb)
            _scale_buf(rows_b, jb)
            _swait(ja, rows_a, sem_sa)

            @pl.when(i < NB2 - 1)
            def _():
                _gstart(ja + 2, rows_a, sem_ga)

            _sstart(jb, rows_b, sem_sb)
            return 0

        lax.fori_loop(0, NB2, _pair, 0)
        _swait(SEGC - 1, rows_b, sem_sb)
        return 0

    lax.fori_loop(0, nseg, _seg, 0)

    plsc.subcore_barrier()

    for i in range(NFLUSH):
        sl = pl.ds(base + i * FCHUNK, FCHUNK)
        buf, sg, ss = ((rows_a, sem_ga, sem_sa) if i % 2 == 0
                       else (rows_b, sem_gb, sem_sb))
        bsl = buf.at[pl.ds(0, FCHUNK)]
        if i >= 2:
            psl = pl.ds(base + (i - 2) * FCHUNK, FCHUNK)
            pltpu.make_async_copy(bsl, out_hbm.at[c].at[psl], ss).wait()
        pltpu.async_copy(agg_sh.at[sl], bsl, sg).wait()
        pltpu.async_copy(bsl, out_hbm.at[c].at[sl], ss)
    for i in (NFLUSH - 2, NFLUSH - 1):
        sl = pl.ds(base + i * FCHUNK, FCHUNK)
        buf, ss = (rows_a, sem_sa) if i % 2 == 0 else (rows_b, sem_sb)
        pltpu.make_async_copy(buf.at[pl.ds(0, FCHUNK)],
                              out_hbm.at[c].at[sl], ss).wait()

    @pl.when(s == NS - 1)
    def _flush_tail():
        sl = pl.ds(NS * STRIP, TAIL)
        tsl = rows_a.at[pl.ds(0, TAIL)]
        pltpu.sync_copy(agg_sh.at[sl], tsl)
        pltpu.sync_copy(tsl, out_hbm.at[c].at[sl])


def _tc_finish_kernel(p_ref, w_ref, b_ref, z_ref, z1_ref, z2_ref):
    agg = p_ref[0] + p_ref[1]
    h = jnp.dot(agg, w_ref[...], preferred_element_type=jnp.float32)
    b = b_ref[...]
    z_ref[...] = jnp.maximum(h + b, 0.0)
    z1_ref[...] = jnp.maximum(h * 0.9 + b, 0.0)
    z2_ref[...] = jnp.maximum(h * 0.8 + b, 0.0)


def kernel(x, edge_index, edge_weight, W, b):
    pad = E_PAD - E
    src = jnp.pad(edge_index[0], (0, pad)).reshape(TOT, C)
    dst = jnp.pad(edge_index[1], (0, pad)).reshape(TOT, C)
    ew = jnp.pad(edge_weight, (0, pad)).reshape(TOT, C)

    mesh = plsc.VectorSubcoreMesh(core_axis_name="c", subcore_axis_name="s")
    sc_call = pl.kernel(
        _sc_agg_kernel,
        out_type=jax.ShapeDtypeStruct((NC, N, D), jnp.float32),
        mesh=mesh,
        scratch_types=[
            pltpu.VMEM((SEGC, C), jnp.int32),
            pltpu.VMEM((SEGC, C), jnp.int32),
            pltpu.VMEM((SEGC, C), jnp.float32),
            pltpu.VMEM((C, D), jnp.float32),
            pltpu.VMEM((C, D), jnp.float32),
            pltpu.VMEM_SHARED((N, D), jnp.float32),
            pltpu.SemaphoreType.DMA,
            pltpu.SemaphoreType.DMA,
            pltpu.SemaphoreType.DMA,
            pltpu.SemaphoreType.DMA,
        ],
    )
    partials = sc_call(src, dst, ew, x)

    R = 1000
    grid = (N // R,)
    z, z1, z2 = pl.pallas_call(
        _tc_finish_kernel,
        grid=grid,
        in_specs=[
            pl.BlockSpec((NC, R, D), lambda i: (0, i, 0)),
            pl.BlockSpec((D, D), lambda i: (0, 0)),
            pl.BlockSpec((1, D), lambda i: (0, 0)),
        ],
        out_specs=[
            pl.BlockSpec((R, D), lambda i: (i, 0)),
            pl.BlockSpec((R, D), lambda i: (i, 0)),
            pl.BlockSpec((R, D), lambda i: (i, 0)),
        ],
        out_shape=[jax.ShapeDtypeStruct((N, D), jnp.float32)] * 3,
    )(partials, W, b.reshape(1, D))
    return (z, z1, z2)
